# Initial kernel scaffold; baseline (speedup 1.0000x reference)
import jax
import jax.numpy as jnp
from jax.experimental import pallas as pl


def kernel(z_sc, z_mp, pos, W1, b1, W2, b2):
    raise NotImplementedError("write your pallas kernel here")


# speedup vs baseline: 3.8989x; 3.8989x over previous
import jax
import jax.numpy as jnp
from jax.experimental import pallas as pl
from jax.experimental.pallas import tpu as pltpu

TAU_ = 0.8
LAMBDA_ = 0.5


def _proj_kernel(z_ref, w1t_ref, b1_ref, w2t_ref, b2_ref, out_ref):
    z = z_ref[...]
    h = jnp.dot(z, w1t_ref[...], preferred_element_type=jnp.float32) + b1_ref[...]
    h = jnp.where(h > 0, h, jnp.exp(h) - 1.0)
    p = jnp.dot(h, w2t_ref[...], preferred_element_type=jnp.float32) + b2_ref[...]
    norm = jnp.sqrt(jnp.sum(p * p, axis=1, keepdims=True))
    out_ref[...] = p / norm


def _loss_kernel(n_valid_ref, qs_ref, qm_ref, ks_ref, km_ref, pos_ref, out_ref,
                 num_sc, den_sc, num_mp, den_mp):
    j = pl.program_id(0)
    nsteps = pl.num_programs(0)
    cblk = pos_ref.shape[1]

    @pl.when(j == 0)
    def _init():
        num_sc[...] = jnp.zeros_like(num_sc)
        den_sc[...] = jnp.zeros_like(den_sc)
        num_mp[...] = jnp.zeros_like(num_mp)
        den_mp[...] = jnp.zeros_like(den_mp)

    inv_tau = jnp.float32(1.0 / TAU_)
    p = pos_ref[...]
    col = jax.lax.broadcasted_iota(jnp.int32, (1, cblk), 1) + j * cblk
    mask = (col < n_valid_ref[0]).astype(jnp.float32)

    dn = (((1,), (1,)), ((), ()))
    e1 = jnp.exp(jax.lax.dot_general(qs_ref[...], km_ref[...], dn,
                                     preferred_element_type=jnp.float32) * inv_tau) * mask
    num_sc[...] += jnp.sum(e1 * p, axis=1, keepdims=True)
    den_sc[...] += jnp.sum(e1, axis=1, keepdims=True)

    e2 = jnp.exp(jax.lax.dot_general(qm_ref[...], ks_ref[...], dn,
                                     preferred_element_type=jnp.float32) * inv_tau) * mask
    num_mp[...] += jnp.sum(e2 * p, axis=1, keepdims=True)
    den_mp[...] += jnp.sum(e2, axis=1, keepdims=True)

    @pl.when(j == nsteps - 1)
    def _fini():
        eps = jnp.float32(1e-8)
        loss_sc = -jnp.mean(jnp.log(num_sc[...] / (den_sc[...] + eps)))
        loss_mp = -jnp.mean(jnp.log(num_mp[...] / (den_mp[...] + eps)))
        loss = jnp.float32(LAMBDA_) * loss_sc + jnp.float32(1.0 - LAMBDA_) * loss_mp
        out_ref[...] = jnp.reshape(loss, (1, 1))


def kernel(z_sc, z_mp, pos, W1, b1, W2, b2):
    N, d = z_sc.shape
    B = pos.shape[0]
    CBLK = 2048
    n_col_blocks = -(-N // CBLK)
    Npad = n_col_blocks * CBLK

    z = jnp.zeros((2 * Npad, d), jnp.float32)
    z = z.at[:N].set(z_sc).at[Npad:Npad + N].set(z_mp)
    pos_pad = jnp.pad(pos, ((0, 0), (0, Npad - N)))

    RBLK = 2048
    zh = pl.pallas_call(
        _proj_kernel,
        grid=(2 * Npad // RBLK,),
        in_specs=[
            pl.BlockSpec((RBLK, d), lambda i: (i, 0)),
            pl.BlockSpec((d, d), lambda i: (0, 0)),
            pl.BlockSpec((1, d), lambda i: (0, 0)),
            pl.BlockSpec((d, d), lambda i: (0, 0)),
            pl.BlockSpec((1, d), lambda i: (0, 0)),
        ],
        out_specs=pl.BlockSpec((RBLK, d), lambda i: (i, 0)),
        out_shape=jax.ShapeDtypeStruct((2 * Npad, d), jnp.float32),
    )(z, W1.T, b1.reshape(1, d), W2.T, b2.reshape(1, d))

    zh_sc = zh[:Npad]
    zh_mp = zh[Npad:]
    qs = zh_sc[:B]
    qm = zh_mp[:B]
    n_valid = jnp.full((1,), N, jnp.int32)

    out = pl.pallas_call(
        _loss_kernel,
        grid=(n_col_blocks,),
        in_specs=[
            pl.BlockSpec(memory_space=pltpu.SMEM),
            pl.BlockSpec((B, d), lambda j: (0, 0)),
            pl.BlockSpec((B, d), lambda j: (0, 0)),
            pl.BlockSpec((CBLK, d), lambda j: (j, 0)),
            pl.BlockSpec((CBLK, d), lambda j: (j, 0)),
            pl.BlockSpec((B, CBLK), lambda j: (0, j)),
        ],
        out_specs=pl.BlockSpec((1, 1), lambda j: (0, 0)),
        out_shape=jax.ShapeDtypeStruct((1, 1), jnp.float32),
        scratch_shapes=[
            pltpu.VMEM((B, 1), jnp.float32),
            pltpu.VMEM((B, 1), jnp.float32),
            pltpu.VMEM((B, 1), jnp.float32),
            pltpu.VMEM((B, 1), jnp.float32),
        ],
    )(n_valid, qs, qm, zh_sc, zh_mp, pos_pad)

    return out[0, 0]


# speedup vs baseline: 4.2459x; 1.0890x over previous
import jax
import jax.numpy as jnp
from jax.experimental import pallas as pl
from jax.experimental.pallas import tpu as pltpu

TAU_ = 0.8
LAMBDA_ = 0.5


def _proj_kernel(nvalid_ref, z_ref, w1t_ref, b1_ref, w2t_ref, b2_ref, out_ref):
    i = pl.program_id(0)
    rblk = z_ref.shape[0]
    z = z_ref[...]
    h = jnp.dot(z, w1t_ref[...], preferred_element_type=jnp.float32) + b1_ref[...]
    h = jnp.where(h > 0, h, jnp.exp(h) - 1.0)
    p = jnp.dot(h, w2t_ref[...], preferred_element_type=jnp.float32) + b2_ref[...]
    norm = jnp.sqrt(jnp.sum(p * p, axis=1, keepdims=True))
    n, npad = nvalid_ref[0], nvalid_ref[1]
    g = jax.lax.broadcasted_iota(jnp.int32, (rblk, 1), 0) + i * rblk
    valid = (g < n) | ((g >= npad) & (g < npad + n))
    out_ref[...] = jnp.where(valid, p / norm, 0.0).astype(out_ref.dtype)


def _loss_kernel(n_valid_ref, qs_ref, qm_ref, ks_ref, km_ref, pos_ref, out_ref,
                 num_sc, den_sc, num_mp, den_mp):
    j = pl.program_id(0)
    nsteps = pl.num_programs(0)
    cblk = pos_ref.shape[1]

    @pl.when(j == 0)
    def _init():
        num_sc[...] = jnp.zeros_like(num_sc)
        den_sc[...] = jnp.zeros_like(den_sc)
        num_mp[...] = jnp.zeros_like(num_mp)
        den_mp[...] = jnp.zeros_like(den_mp)

    inv_tau = jnp.float32(1.0 / TAU_)
    p = pos_ref[...]

    dn = (((1,), (1,)), ((), ()))
    e1 = jnp.exp(jax.lax.dot_general(qs_ref[...], km_ref[...], dn,
                                     preferred_element_type=jnp.float32) * inv_tau)
    num_sc[...] += jnp.sum(e1 * p, axis=1, keepdims=True)
    den_sc[...] += jnp.sum(e1, axis=1, keepdims=True)

    e2 = jnp.exp(jax.lax.dot_general(qm_ref[...], ks_ref[...], dn,
                                     preferred_element_type=jnp.float32) * inv_tau)
    num_mp[...] += jnp.sum(e2 * p, axis=1, keepdims=True)
    den_mp[...] += jnp.sum(e2, axis=1, keepdims=True)

    @pl.when(j == nsteps - 1)
    def _fini():
        pad = (nsteps * cblk - n_valid_ref[0]).astype(jnp.float32)
        eps = jnp.float32(1e-8)
        loss_sc = -jnp.mean(jnp.log(num_sc[...] / (den_sc[...] - pad + eps)))
        loss_mp = -jnp.mean(jnp.log(num_mp[...] / (den_mp[...] - pad + eps)))
        loss = jnp.float32(LAMBDA_) * loss_sc + jnp.float32(1.0 - LAMBDA_) * loss_mp
        out_ref[...] = jnp.reshape(loss, (1, 1))


def kernel(z_sc, z_mp, pos, W1, b1, W2, b2):
    N, d = z_sc.shape
    B = pos.shape[0]
    CBLK = 2048
    n_col_blocks = -(-N // CBLK)
    Npad = n_col_blocks * CBLK

    z = jnp.zeros((2 * Npad, d), jnp.float32)
    z = z.at[:N].set(z_sc).at[Npad:Npad + N].set(z_mp)
    pos_pad = jnp.pad(pos, ((0, 0), (0, Npad - N)))

    RBLK = 2048
    nvalid2 = jnp.array([N, Npad], jnp.int32)
    zh = pl.pallas_call(
        _proj_kernel,
        grid=(2 * Npad // RBLK,),
        in_specs=[
            pl.BlockSpec(memory_space=pltpu.SMEM),
            pl.BlockSpec((RBLK, d), lambda i: (i, 0)),
            pl.BlockSpec((d, d), lambda i: (0, 0)),
            pl.BlockSpec((1, d), lambda i: (0, 0)),
            pl.BlockSpec((d, d), lambda i: (0, 0)),
            pl.BlockSpec((1, d), lambda i: (0, 0)),
        ],
        out_specs=pl.BlockSpec((RBLK, d), lambda i: (i, 0)),
        out_shape=jax.ShapeDtypeStruct((2 * Npad, d), jnp.bfloat16),
    )(nvalid2, z, W1.T, b1.reshape(1, d), W2.T, b2.reshape(1, d))

    zh_sc = zh[:Npad]
    zh_mp = zh[Npad:]
    qs = zh_sc[:B]
    qm = zh_mp[:B]
    n_valid = jnp.full((1,), N, jnp.int32)

    out = pl.pallas_call(
        _loss_kernel,
        grid=(n_col_blocks,),
        in_specs=[
            pl.BlockSpec(memory_space=pltpu.SMEM),
            pl.BlockSpec((B, d), lambda j: (0, 0)),
            pl.BlockSpec((B, d), lambda j: (0, 0)),
            pl.BlockSpec((CBLK, d), lambda j: (j, 0)),
            pl.BlockSpec((CBLK, d), lambda j: (j, 0)),
            pl.BlockSpec((B, CBLK), lambda j: (0, j)),
        ],
        out_specs=pl.BlockSpec((1, 1), lambda j: (0, 0)),
        out_shape=jax.ShapeDtypeStruct((1, 1), jnp.float32),
        scratch_shapes=[
            pltpu.VMEM((B, 1), jnp.float32),
            pltpu.VMEM((B, 1), jnp.float32),
            pltpu.VMEM((B, 1), jnp.float32),
            pltpu.VMEM((B, 1), jnp.float32),
        ],
    )(n_valid, qs, qm, zh_sc, zh_mp, pos_pad)

    return out[0, 0]


# speedup vs baseline: 7.0148x; 1.6521x over previous
import jax
import jax.numpy as jnp
from jax.experimental import pallas as pl
from jax.experimental.pallas import tpu as pltpu

TAU_ = 0.8
LAMBDA_ = 0.5
LN2_ = 0.6931471805599453


def _projnorm(z, w1t, b1, w2t, b2, row0, n):
    h = jnp.dot(z, w1t, preferred_element_type=jnp.float32) + b1
    h = jnp.where(h > 0, h, jnp.exp(h) - 1.0)
    p = jnp.dot(h, w2t, preferred_element_type=jnp.float32) + b2
    norm = jnp.sqrt(jnp.sum(p * p, axis=1, keepdims=True))
    g = jax.lax.broadcasted_iota(jnp.int32, (z.shape[0], 1), 0) + row0
    return jnp.where(g < n, p / norm, 0.0)


def _make_loss_kernel(n, b):
    def _loss_kernel(zsc_ref, zmp_ref, w1t_ref, b1_ref, w2t_ref, b2_ref,
                     pos_ref, out_ref, qs_ref, qm_ref,
                     num_sc, den_sc, num_mp, den_mp):
        j = pl.program_id(0)
        nsteps = pl.num_programs(0)
        cblk = pos_ref.shape[1]
        row0 = j * cblk

        w1t, b1 = w1t_ref[...], b1_ref[...]
        w2t, b2 = w2t_ref[...], b2_ref[...]
        ks = _projnorm(zsc_ref[...], w1t, b1, w2t, b2, row0, n)
        km = _projnorm(zmp_ref[...], w1t, b1, w2t, b2, row0, n)
        ks16 = ks.astype(jnp.bfloat16)
        km16 = km.astype(jnp.bfloat16)

        @pl.when(j == 0)
        def _init():
            scale = jnp.float32(1.0 / (TAU_ * LN2_))
            qs_ref[...] = (ks[:b] * scale).astype(jnp.bfloat16)
            qm_ref[...] = (km[:b] * scale).astype(jnp.bfloat16)
            zero = jnp.zeros((b, 1), jnp.float32)
            num_sc[...] = zero
            den_sc[...] = zero
            num_mp[...] = zero
            den_mp[...] = zero

        col = jax.lax.broadcasted_iota(jnp.int32, (b, cblk), 1) + row0
        p = jnp.where(col < n, pos_ref[...], 0.0)

        dn = (((1,), (1,)), ((), ()))
        e1 = jnp.exp2(jax.lax.dot_general(qs_ref[...], km16, dn,
                                          preferred_element_type=jnp.float32))
        num_sc[...] += jnp.sum(e1 * p, axis=1, keepdims=True)
        den_sc[...] += jnp.sum(e1, axis=1, keepdims=True)

        e2 = jnp.exp2(jax.lax.dot_general(qm_ref[...], ks16, dn,
                                          preferred_element_type=jnp.float32))
        num_mp[...] += jnp.sum(e2 * p, axis=1, keepdims=True)
        den_mp[...] += jnp.sum(e2, axis=1, keepdims=True)

        @pl.when(j == nsteps - 1)
        def _fini():
            pad = jnp.float32(nsteps * cblk - n)
            eps = jnp.float32(1e-8)
            loss_sc = -jnp.mean(jnp.log(num_sc[...] / (den_sc[...] - pad + eps)))
            loss_mp = -jnp.mean(jnp.log(num_mp[...] / (den_mp[...] - pad + eps)))
            loss = jnp.float32(LAMBDA_) * loss_sc + jnp.float32(1.0 - LAMBDA_) * loss_mp
            out_ref[...] = jnp.reshape(loss, (1, 1))

    return _loss_kernel


def kernel(z_sc, z_mp, pos, W1, b1, W2, b2):
    N, d = z_sc.shape
    B = pos.shape[0]
    CBLK = 2048
    n_blocks = -(-N // CBLK)

    out = pl.pallas_call(
        _make_loss_kernel(N, B),
        grid=(n_blocks,),
        in_specs=[
            pl.BlockSpec((CBLK, d), lambda j: (j, 0)),
            pl.BlockSpec((CBLK, d), lambda j: (j, 0)),
            pl.BlockSpec((d, d), lambda j: (0, 0)),
            pl.BlockSpec((1, d), lambda j: (0, 0)),
            pl.BlockSpec((d, d), lambda j: (0, 0)),
            pl.BlockSpec((1, d), lambda j: (0, 0)),
            pl.BlockSpec((B, CBLK), lambda j: (0, j)),
        ],
        out_specs=pl.BlockSpec((1, 1), lambda j: (0, 0)),
        out_shape=jax.ShapeDtypeStruct((1, 1), jnp.float32),
        scratch_shapes=[
            pltpu.VMEM((B, d), jnp.bfloat16),
            pltpu.VMEM((B, d), jnp.bfloat16),
            pltpu.VMEM((B, 1), jnp.float32),
            pltpu.VMEM((B, 1), jnp.float32),
            pltpu.VMEM((B, 1), jnp.float32),
            pltpu.VMEM((B, 1), jnp.float32),
        ],
    )(z_sc, z_mp, W1.T, b1.reshape(1, d), W2.T, b2.reshape(1, d), pos)

    return out[0, 0]


# speedup vs baseline: 7.3118x; 1.0423x over previous
import jax
import jax.numpy as jnp
from jax.experimental import pallas as pl
from jax.experimental.pallas import tpu as pltpu

TAU_ = 0.8
LAMBDA_ = 0.5
LN2_ = 0.6931471805599453


def _make_loss_kernel(n, npad, b, rb):
    def _projnorm(z, w1t, b1, w2t, b2):
        h = jnp.dot(z, w1t, preferred_element_type=jnp.float32) + b1
        h = jnp.where(h > 0, h, jnp.exp(h) - 1.0)
        p = jnp.dot(h, w2t, preferred_element_type=jnp.float32) + b2
        norm = jnp.sqrt(jnp.sum(p * p, axis=1, keepdims=True))
        return p / norm

    def _loss_kernel(zsc_ref, zmp_ref, w1t_ref, b1_ref, w2t_ref, b2_ref,
                     pos_ref, out_ref, khs_ref, khm_ref):
        i = pl.program_id(0)
        nsteps = pl.num_programs(0)

        @pl.when(i == 0)
        def _init():
            w1t, b1 = w1t_ref[...], b1_ref[...]
            w2t, b2 = w2t_ref[...], b2_ref[...]
            scale = jnp.float32(1.0 / (TAU_ * LN2_))
            zh_sc = _projnorm(zsc_ref[...], w1t, b1, w2t, b2) * scale
            zh_mp = _projnorm(zmp_ref[...], w1t, b1, w2t, b2)
            khs_ref[pl.ds(0, n), :] = zh_sc.astype(jnp.bfloat16)
            khm_ref[pl.ds(0, n), :] = zh_mp.astype(jnp.bfloat16)
            pad16 = jnp.zeros((npad - n, khs_ref.shape[1]), jnp.bfloat16)
            khs_ref[pl.ds(n, npad - n), :] = pad16
            khm_ref[pl.ds(n, npad - n), :] = pad16
            out_ref[...] = jnp.zeros((1, 1), jnp.float32)

        qs = khs_ref[pl.ds(i * rb, rb), :]
        qm = khm_ref[pl.ds(i * rb, rb), :]
        p = pos_ref[...]

        dn = (((1,), (1,)), ((), ()))
        pad = jnp.float32(npad - n)
        eps = jnp.float32(1e-8)

        e1 = jnp.exp2(jax.lax.dot_general(qs, khm_ref[...], dn,
                                          preferred_element_type=jnp.float32))
        num1 = jnp.sum(e1[:, :n] * p, axis=1, keepdims=True)
        den1 = jnp.sum(e1, axis=1, keepdims=True) - pad
        e2 = jnp.exp2(jax.lax.dot_general(qm, khs_ref[...], dn,
                                          preferred_element_type=jnp.float32))
        num2 = jnp.sum(e2[:, :n] * p, axis=1, keepdims=True)
        den2 = jnp.sum(e2, axis=1, keepdims=True) - pad

        c1 = jnp.float32(-LAMBDA_ / b)
        c2 = jnp.float32(-(1.0 - LAMBDA_) / b)
        part = (c1 * jnp.sum(jnp.log(num1 / (den1 + eps)))
                + c2 * jnp.sum(jnp.log(num2 / (den2 + eps))))
        out_ref[...] += jnp.reshape(part, (1, 1))

    return _loss_kernel


def kernel(z_sc, z_mp, pos, W1, b1, W2, b2):
    N, d = z_sc.shape
    B = pos.shape[0]
    Npad = -(-N // 128) * 128
    RB = 128
    n_blocks = B // RB

    out = pl.pallas_call(
        _make_loss_kernel(N, Npad, B, RB),
        grid=(n_blocks,),
        in_specs=[
            pl.BlockSpec((N, d), lambda i: (0, 0)),
            pl.BlockSpec((N, d), lambda i: (0, 0)),
            pl.BlockSpec((d, d), lambda i: (0, 0)),
            pl.BlockSpec((1, d), lambda i: (0, 0)),
            pl.BlockSpec((d, d), lambda i: (0, 0)),
            pl.BlockSpec((1, d), lambda i: (0, 0)),
            pl.BlockSpec((RB, N), lambda i: (i, 0)),
        ],
        out_specs=pl.BlockSpec((1, 1), lambda i: (0, 0)),
        out_shape=jax.ShapeDtypeStruct((1, 1), jnp.float32),
        scratch_shapes=[
            pltpu.VMEM((Npad, d), jnp.bfloat16),
            pltpu.VMEM((Npad, d), jnp.bfloat16),
        ],
    )(z_sc, z_mp, W1.T, b1.reshape(1, d), W2.T, b2.reshape(1, d), pos)

    return out[0, 0]


# speedup vs baseline: 8.0764x; 1.1046x over previous
import jax
import jax.numpy as jnp
from jax.experimental import pallas as pl
from jax.experimental.pallas import tpu as pltpu

TAU_ = 0.8
LAMBDA_ = 0.5
LN2_ = 0.6931471805599453


def _make_loss_kernel(n, npad, b, rb):
    def _projnorm(z, w1t, b1, w2t, b2):
        h = jnp.dot(z, w1t, preferred_element_type=jnp.float32) + b1
        h = jnp.where(h > 0, h, jnp.exp(h) - 1.0)
        p = jnp.dot(h, w2t, preferred_element_type=jnp.float32) + b2
        norm = jnp.sqrt(jnp.sum(p * p, axis=1, keepdims=True))
        return p / norm

    def _loss_kernel(zsc_ref, zmp_ref, w1t_ref, b1_ref, w2t_ref, b2_ref,
                     pos_ref, out_ref, khs_ref, khm_ref):
        i = pl.program_id(0)
        nsteps = pl.num_programs(0)

        @pl.when(i == 0)
        def _init():
            w1t, b1 = w1t_ref[...], b1_ref[...]
            w2t, b2 = w2t_ref[...], b2_ref[...]
            scale = jnp.float32(1.0 / (TAU_ * LN2_))
            zh_sc = _projnorm(zsc_ref[...], w1t, b1, w2t, b2) * scale
            zh_mp = _projnorm(zmp_ref[...], w1t, b1, w2t, b2)
            khs_ref[pl.ds(0, n), :] = zh_sc.astype(jnp.bfloat16)
            khm_ref[pl.ds(0, n), :] = zh_mp.astype(jnp.bfloat16)
            pad16 = jnp.zeros((npad - n, khs_ref.shape[1]), jnp.bfloat16)
            khs_ref[pl.ds(n, npad - n), :] = pad16
            khm_ref[pl.ds(n, npad - n), :] = pad16
            out_ref[...] = jnp.zeros((1, 1), jnp.float32)

        qs = khs_ref[pl.ds(i * rb, rb), :]
        qm = khm_ref[pl.ds(i * rb, rb), :]
        p = pos_ref[...].astype(jnp.float32)

        dn = (((1,), (1,)), ((), ()))
        pad = jnp.float32(npad - n)
        eps = jnp.float32(1e-8)

        e1 = jnp.exp2(jax.lax.dot_general(qs, khm_ref[...], dn,
                                          preferred_element_type=jnp.float32))
        num1 = jnp.sum(e1[:, :n] * p, axis=1, keepdims=True)
        den1 = jnp.sum(e1, axis=1, keepdims=True) - pad
        e2 = jnp.exp2(jax.lax.dot_general(qm, khs_ref[...], dn,
                                          preferred_element_type=jnp.float32))
        num2 = jnp.sum(e2[:, :n] * p, axis=1, keepdims=True)
        den2 = jnp.sum(e2, axis=1, keepdims=True) - pad

        c1 = jnp.float32(-LAMBDA_ / b)
        c2 = jnp.float32(-(1.0 - LAMBDA_) / b)
        part = (c1 * jnp.sum(jnp.log(num1 / (den1 + eps)))
                + c2 * jnp.sum(jnp.log(num2 / (den2 + eps))))
        out_ref[...] += jnp.reshape(part, (1, 1))

    return _loss_kernel


def kernel(z_sc, z_mp, pos, W1, b1, W2, b2):
    N, d = z_sc.shape
    B = pos.shape[0]
    Npad = -(-N // 128) * 128
    RB = 128
    n_blocks = B // RB

    out = pl.pallas_call(
        _make_loss_kernel(N, Npad, B, RB),
        grid=(n_blocks,),
        in_specs=[
            pl.BlockSpec((N, d), lambda i: (0, 0)),
            pl.BlockSpec((N, d), lambda i: (0, 0)),
            pl.BlockSpec((d, d), lambda i: (0, 0)),
            pl.BlockSpec((1, d), lambda i: (0, 0)),
            pl.BlockSpec((d, d), lambda i: (0, 0)),
            pl.BlockSpec((1, d), lambda i: (0, 0)),
            pl.BlockSpec((RB, N), lambda i: (i, 0)),
        ],
        out_specs=pl.BlockSpec((1, 1), lambda i: (0, 0)),
        out_shape=jax.ShapeDtypeStruct((1, 1), jnp.float32),
        scratch_shapes=[
            pltpu.VMEM((Npad, d), jnp.bfloat16),
            pltpu.VMEM((Npad, d), jnp.bfloat16),
        ],
    )(z_sc, z_mp, W1.T, b1.reshape(1, d), W2.T, b2.reshape(1, d),
      pos.astype(jnp.int8))

    return out[0, 0]
